# Initial kernel scaffold; baseline (speedup 1.0000x reference)
import jax
import jax.numpy as jnp
from jax.experimental import pallas as pl


def kernel(fea_prev, fea_cur, fea_next, xyz_prev, xyz_cur, xyz_next, batch, p_w1, p_b1, p_g1, p_be1, p_w2, p_b2, q_w, q_b, k_w, k_b, v_w, v_b, w_g1, w_be1, w_w, w_wb, w_g2, w_be2):
    raise NotImplementedError("write your pallas kernel here")


# speedup vs baseline: 4.7419x; 4.7419x over previous
import functools

import jax
import jax.numpy as jnp
from jax import lax
from jax.experimental import pallas as pl
from jax.experimental.pallas import tpu as pltpu
from jax.experimental.pallas import tpu_sc as plsc

_K = 16
_EPS = 1e-5
_SLOPE = 0.01
_INTERP = False


def _leaky(x):
    return jnp.where(x >= 0, x, _SLOPE * x)


def _rep16(x):
    m, c = x.shape
    return jnp.broadcast_to(x[:, None, :], (m, _K, c)).reshape(m * _K, c)


def _mm_bias(x, w, b, bm=1024):
    m, kd = x.shape
    nd = w.shape[1]

    def body(x_ref, w_ref, b_ref, o_ref):
        o_ref[...] = (
            jnp.dot(x_ref[...], w_ref[...], preferred_element_type=jnp.float32)
            + b_ref[...]
        )

    return pl.pallas_call(
        body,
        grid=(m // bm,),
        in_specs=[
            pl.BlockSpec((bm, kd), lambda i: (i, 0)),
            pl.BlockSpec((kd, nd), lambda i: (0, 0)),
            pl.BlockSpec((1, nd), lambda i: (0, 0)),
        ],
        out_specs=pl.BlockSpec((bm, nd), lambda i: (i, 0)),
        out_shape=jax.ShapeDtypeStruct((m, nd), jnp.float32),
        interpret=_INTERP,
    )(x, w, b.reshape(1, -1))


def _knn(yq8, xkt, tq=256):
    nq = yq8.shape[0]
    nk = xkt.shape[1]

    def body(yq_ref, xk_ref, o_ref, dscr):
        yq = yq_ref[...]
        xk = xk_ref[...]
        ny = jnp.sum(yq * yq, axis=1, keepdims=True)
        nx = jnp.sum(xk * xk, axis=0, keepdims=True)
        d = ny + nx - 2.0 * jnp.dot(yq, xk, preferred_element_type=jnp.float32)
        dscr[...] = d
        li = lax.broadcasted_iota(jnp.int32, (tq, _K), 1)
        big = jnp.int32(0x7FFFFFFF)

        def it(j, acc):
            dv = dscr[...]
            col = lax.broadcasted_iota(jnp.int32, dv.shape, 1)
            mn = jnp.min(dv, axis=1)
            cand = jnp.where(dv == mn[:, None], col, big)
            c = jnp.min(cand, axis=1)
            dscr[...] = jnp.where(col == c[:, None], jnp.float32(jnp.inf), dv)
            return jnp.where(li == j, c[:, None], acc)

        o_ref[...] = lax.fori_loop(0, _K, it, jnp.zeros((tq, _K), jnp.int32))

    return pl.pallas_call(
        body,
        grid=(nq // tq,),
        in_specs=[
            pl.BlockSpec((tq, 8), lambda i: (i, 0)),
            pl.BlockSpec((8, nk), lambda i: (0, 0)),
        ],
        out_specs=pl.BlockSpec((tq, _K), lambda i: (i, 0)),
        out_shape=jax.ShapeDtypeStruct((nq, _K), jnp.int32),
        scratch_shapes=[pltpu.VMEM((tq, nk), jnp.float32)],
        interpret=_INTERP,
    )(yq8, xkt)


def _sc_gather(table, idx):
    nk = idx.shape[0]
    d = table.shape[1]
    ncores, nsub = 2, 16
    nw = ncores * nsub
    bpw = nk // nw
    ch = 128
    nch = bpw // ch
    mesh = plsc.VectorSubcoreMesh(core_axis_name="c", subcore_axis_name="s")

    @functools.partial(
        pl.kernel,
        mesh=mesh,
        out_type=jax.ShapeDtypeStruct((nk, d), jnp.float32),
        scratch_types=[
            pltpu.VMEM((ch,), jnp.int32),
            pltpu.VMEM((ch, d), jnp.float32),
            pltpu.SemaphoreType.DMA,
        ],
    )
    def k(table_hbm, idx_hbm, out_hbm, idx_v, rows_v, sem):
        wid = lax.axis_index("s") * ncores + lax.axis_index("c")
        base = wid * bpw

        def body(c, carry):
            off = base + c * ch
            pltpu.sync_copy(idx_hbm.at[pl.ds(off, ch)], idx_v)
            pltpu.async_copy(table_hbm.at[idx_v], rows_v, sem).wait()
            pltpu.sync_copy(rows_v, out_hbm.at[pl.ds(off, ch)])
            return carry

        lax.fori_loop(0, nch, body, 0)

    return k(table, idx)


def _stats_a(g, xqr, xoff, bm=2048):
    nk, gd = g.shape
    grid = nk // bm

    def body(g_ref, xqr_ref, m1_ref, m2_ref):
        @pl.when(pl.program_id(0) == 0)
        def _():
            m1_ref[...] = jnp.zeros_like(m1_ref)
            m2_ref[...] = jnp.zeros_like(m2_ref)

        t = g_ref[:, xoff:xoff + _K] - xqr_ref[...]
        m1_ref[0:1, :] += jnp.sum(t, axis=0, keepdims=True)
        m2_ref[...] += lax.dot_general(
            t, t, (((0,), (0,)), ((), ())), preferred_element_type=jnp.float32
        )

    return pl.pallas_call(
        body,
        grid=(grid,),
        in_specs=[
            pl.BlockSpec((bm, gd), lambda i: (i, 0)),
            pl.BlockSpec((bm, _K), lambda i: (i, 0)),
        ],
        out_specs=[
            pl.BlockSpec((8, _K), lambda i: (0, 0)),
            pl.BlockSpec((_K, _K), lambda i: (0, 0)),
        ],
        out_shape=[
            jax.ShapeDtypeStruct((8, _K), jnp.float32),
            jax.ShapeDtypeStruct((_K, _K), jnp.float32),
        ],
        interpret=_INTERP,
    )(g, xqr)


def _pass_b(g, qt, xqr, m1, m2, w1p, pb1, pg1, pbe1, pw2, pb2, cnt, bm=2048):
    nk = g.shape[0]
    co = pw2.shape[1]
    grid = nk // bm
    bq = bm // _K
    inv = 1.0 / cnt

    def body(g_ref, qt_ref, xqr_ref, m1_ref, m2_ref, w1_ref, b1_ref, g1_ref,
             be1_ref, w2_ref, b2_ref, w_out, vf_out, sw_ref, sww_ref):
        @pl.when(pl.program_id(0) == 0)
        def _():
            sw_ref[...] = jnp.zeros_like(sw_ref)
            sww_ref[...] = jnp.zeros_like(sww_ref)

        gv = g_ref[...]
        kpg = gv[:, :co]
        vpg = gv[:, co:2 * co]
        xg = gv[:, 2 * co:2 * co + _K]
        w1 = w1_ref[...]
        b1 = b1_ref[...]
        md = m1_ref[0:1, :] * inv
        mu = jnp.dot(md, w1, preferred_element_type=jnp.float32) + b1
        t2 = jnp.dot(m2_ref[...] * inv, w1, preferred_element_type=jnp.float32)
        e2 = jnp.sum(w1 * t2, axis=0, keepdims=True) + b1 * (2.0 * mu - b1)
        var = e2 - mu * mu
        sc = g1_ref[...] / jnp.sqrt(var + _EPS)
        sh = be1_ref[...] - mu * sc

        dd = xg - xqr_ref[...]
        pre = jnp.dot(dd, w1, preferred_element_type=jnp.float32) + b1
        ph = _leaky(pre * sc + sh)
        pe = jnp.dot(ph, w2_ref[...], preferred_element_type=jnp.float32) + b2_ref[...]

        qtb = _rep16(qt_ref[...])
        w = qtb - kpg + pe
        vf = vpg + pe
        w_out[...] = w
        vf_out[...] = vf
        sw_ref[0:1, :] += jnp.sum(w, axis=0, keepdims=True)
        sww_ref[0:1, :] += jnp.sum(w * w, axis=0, keepdims=True)

    return pl.pallas_call(
        body,
        grid=(grid,),
        in_specs=[
            pl.BlockSpec((bm, g.shape[1]), lambda i: (i, 0)),
            pl.BlockSpec((bq, co), lambda i: (i, 0)),
            pl.BlockSpec((bm, _K), lambda i: (i, 0)),
            pl.BlockSpec((8, _K), lambda i: (0, 0)),
            pl.BlockSpec((_K, _K), lambda i: (0, 0)),
            pl.BlockSpec((_K, 64), lambda i: (0, 0)),
            pl.BlockSpec((1, 64), lambda i: (0, 0)),
            pl.BlockSpec((1, 64), lambda i: (0, 0)),
            pl.BlockSpec((1, 64), lambda i: (0, 0)),
            pl.BlockSpec((64, co), lambda i: (0, 0)),
            pl.BlockSpec((1, co), lambda i: (0, 0)),
        ],
        out_specs=[
            pl.BlockSpec((bm, co), lambda i: (i, 0)),
            pl.BlockSpec((bm, co), lambda i: (i, 0)),
            pl.BlockSpec((8, co), lambda i: (0, 0)),
            pl.BlockSpec((8, co), lambda i: (0, 0)),
        ],
        out_shape=[
            jax.ShapeDtypeStruct((nk, co), jnp.float32),
            jax.ShapeDtypeStruct((nk, co), jnp.float32),
            jax.ShapeDtypeStruct((8, co), jnp.float32),
            jax.ShapeDtypeStruct((8, co), jnp.float32),
        ],
        interpret=_INTERP,
    )(g, qt, xqr, m1, m2, w1p, pb1.reshape(1, -1), pg1.reshape(1, -1),
      pbe1.reshape(1, -1), pw2, pb2.reshape(1, -1))


def _pass_c(w, a1, b1, ww, wb, bm=2048):
    nk, co = w.shape
    grid = nk // bm

    def body(w_ref, a_ref, b_ref, ww_ref, wb_ref, u_out, su_ref, suu_ref):
        @pl.when(pl.program_id(0) == 0)
        def _():
            su_ref[...] = jnp.zeros_like(su_ref)
            suu_ref[...] = jnp.zeros_like(suu_ref)

        h = _leaky(w_ref[...] * a_ref[...] + b_ref[...])
        u = jnp.dot(h, ww_ref[...], preferred_element_type=jnp.float32) + wb_ref[...]
        u_out[...] = u
        su_ref[0:1, :] += jnp.sum(u, axis=0, keepdims=True)
        suu_ref[0:1, :] += jnp.sum(u * u, axis=0, keepdims=True)

    return pl.pallas_call(
        body,
        grid=(grid,),
        in_specs=[
            pl.BlockSpec((bm, co), lambda i: (i, 0)),
            pl.BlockSpec((1, co), lambda i: (0, 0)),
            pl.BlockSpec((1, co), lambda i: (0, 0)),
            pl.BlockSpec((co, co), lambda i: (0, 0)),
            pl.BlockSpec((1, co), lambda i: (0, 0)),
        ],
        out_specs=[
            pl.BlockSpec((bm, co), lambda i: (i, 0)),
            pl.BlockSpec((8, co), lambda i: (0, 0)),
            pl.BlockSpec((8, co), lambda i: (0, 0)),
        ],
        out_shape=[
            jax.ShapeDtypeStruct((nk, co), jnp.float32),
            jax.ShapeDtypeStruct((8, co), jnp.float32),
            jax.ShapeDtypeStruct((8, co), jnp.float32),
        ],
        interpret=_INTERP,
    )(w, a1, b1, ww, wb)


def _pass_d(u, vf, a2, b2, bq=128):
    nk, co = u.shape
    n = nk // _K
    grid = n // bq
    u3 = u.reshape(n, _K, co)
    vf3 = vf.reshape(n, _K, co)

    def body(u_ref, vf_ref, a_ref, b_ref, o_ref):
        a = a_ref[...].reshape(1, 1, co)
        b = b_ref[...].reshape(1, 1, co)
        s = _leaky(u_ref[...] * a + b)
        mx = jnp.max(s, axis=1, keepdims=True)
        e = jnp.exp(s - mx)
        z = jnp.sum(e, axis=1, keepdims=True)
        o_ref[...] = jnp.sum((e / z) * vf_ref[...], axis=1)

    return pl.pallas_call(
        body,
        grid=(grid,),
        in_specs=[
            pl.BlockSpec((bq, _K, co), lambda i: (i, 0, 0)),
            pl.BlockSpec((bq, _K, co), lambda i: (i, 0, 0)),
            pl.BlockSpec((1, co), lambda i: (0, 0)),
            pl.BlockSpec((1, co), lambda i: (0, 0)),
        ],
        out_specs=pl.BlockSpec((bq, co), lambda i: (i, 0)),
        out_shape=jax.ShapeDtypeStruct((n, co), jnp.float32),
        interpret=_INTERP,
    )(u3, vf3, a2, b2)


def kernel(fea_prev, fea_cur, fea_next, xyz_prev, xyz_cur, xyz_next, batch,
           p_w1, p_b1, p_g1, p_be1, p_w2, p_b2,
           q_w, q_b, k_w, k_b, v_w, v_b,
           w_g1, w_be1, w_w, w_wb, w_g2, w_be2):
    n, ci = fea_cur.shape
    co = q_w.shape[1]
    cnt = float(n * _K)

    qt = _mm_bias(fea_cur, q_w, q_b)
    kvw = jnp.concatenate([k_w, v_w], axis=1)
    kvb = jnp.concatenate([k_b, v_b])
    tab_p = _mm_bias(fea_prev, kvw, kvb)
    tab_n = _mm_bias(fea_next, kvw, kvb)

    xq8 = jnp.pad(xyz_cur, ((0, 0), (0, 5)))
    xq16 = jnp.pad(xyz_cur, ((0, 0), (0, _K - 3)))
    xqr = _rep16(xq16)
    w1p = jnp.pad(p_w1, ((0, _K - 3), (0, 0)))

    outs = []
    for xyz_kv, tab in ((xyz_prev, tab_p), (xyz_next, tab_n)):
        xkt = jnp.pad(xyz_kv, ((0, 0), (0, 5))).T
        idx = _knn(xq8, xkt)
        x128 = jnp.pad(xyz_kv, ((0, 0), (0, 125)))
        table = jnp.concatenate([tab, x128], axis=1)
        g = _sc_gather(table, idx.reshape(-1))
        m1, m2 = _stats_a(g, xqr, 2 * co)
        w, vf, sw, sww = _pass_b(g, qt, xqr, m1, m2, w1p,
                                 p_b1, p_g1, p_be1, p_w2, p_b2, cnt)
        swt = jnp.sum(sw, axis=0, keepdims=True)
        swwt = jnp.sum(sww, axis=0, keepdims=True)
        mean_w = swt / cnt
        var_w = swwt / cnt - mean_w * mean_w
        a1 = w_g1.reshape(1, -1) / jnp.sqrt(var_w + _EPS)
        b1 = w_be1.reshape(1, -1) - mean_w * a1
        u, su, suu = _pass_c(w, a1, b1, w_w, w_wb.reshape(1, -1))
        sut = jnp.sum(su, axis=0, keepdims=True)
        suut = jnp.sum(suu, axis=0, keepdims=True)
        mean_u = sut / cnt
        var_u = suut / cnt - mean_u * mean_u
        a2 = w_g2.reshape(1, -1) / jnp.sqrt(var_u + _EPS)
        b2 = w_be2.reshape(1, -1) - mean_u * a2
        outs.append(_pass_d(u, vf, a2, b2))
    return (outs[0], outs[1])


# speedup vs baseline: 4.9094x; 1.0353x over previous
import functools

import jax
import jax.numpy as jnp
from jax import lax
from jax.experimental import pallas as pl
from jax.experimental.pallas import tpu as pltpu
from jax.experimental.pallas import tpu_sc as plsc

_K = 16
_EPS = 1e-5
_SLOPE = 0.01
_INTERP = False


def _leaky(x):
    return jnp.where(x >= 0, x, _SLOPE * x)


def _rep16(x):
    m, c = x.shape
    return jnp.broadcast_to(x[:, None, :], (m, _K, c)).reshape(m * _K, c)


def _mm_bias(x, w, b, bm=1024):
    m, kd = x.shape
    nd = w.shape[1]

    def body(x_ref, w_ref, b_ref, o_ref):
        o_ref[...] = (
            jnp.dot(x_ref[...], w_ref[...], preferred_element_type=jnp.float32)
            + b_ref[...]
        )

    return pl.pallas_call(
        body,
        grid=(m // bm,),
        in_specs=[
            pl.BlockSpec((bm, kd), lambda i: (i, 0)),
            pl.BlockSpec((kd, nd), lambda i: (0, 0)),
            pl.BlockSpec((1, nd), lambda i: (0, 0)),
        ],
        out_specs=pl.BlockSpec((bm, nd), lambda i: (i, 0)),
        out_shape=jax.ShapeDtypeStruct((m, nd), jnp.float32),
        interpret=_INTERP,
    )(x, w, b.reshape(1, -1))


def _knn(yq8, xkt, tq=256):
    nq = yq8.shape[0]
    nk = xkt.shape[1]

    def body(yq_ref, xk_ref, o_ref, dscr):
        yq = yq_ref[...]
        xk = xk_ref[...]
        ny = jnp.sum(yq * yq, axis=1, keepdims=True)
        nx = jnp.sum(xk * xk, axis=0, keepdims=True)
        d = ny + nx - 2.0 * jnp.dot(yq, xk, preferred_element_type=jnp.float32)
        dscr[...] = d
        li = lax.broadcasted_iota(jnp.int32, (tq, _K), 1)
        big = jnp.int32(0x7FFFFFFF)

        def it(j, acc):
            dv = dscr[...]
            col = lax.broadcasted_iota(jnp.int32, dv.shape, 1)
            mn = jnp.min(dv, axis=1)
            cand = jnp.where(dv == mn[:, None], col, big)
            c = jnp.min(cand, axis=1)
            dscr[...] = jnp.where(col == c[:, None], jnp.float32(jnp.inf), dv)
            return jnp.where(li == j, c[:, None], acc)

        o_ref[...] = lax.fori_loop(0, _K, it, jnp.zeros((tq, _K), jnp.int32))

    return pl.pallas_call(
        body,
        grid=(nq // tq,),
        in_specs=[
            pl.BlockSpec((tq, 8), lambda i: (i, 0)),
            pl.BlockSpec((8, nk), lambda i: (0, 0)),
        ],
        out_specs=pl.BlockSpec((tq, _K), lambda i: (i, 0)),
        out_shape=jax.ShapeDtypeStruct((nq, _K), jnp.int32),
        scratch_shapes=[pltpu.VMEM((tq, nk), jnp.float32)],
        interpret=_INTERP,
    )(yq8, xkt)


def _sc_gather(table, idx):
    nk = idx.shape[0]
    d = table.shape[1]
    ncores, nsub = 2, 16
    nw = ncores * nsub
    bpw = nk // nw
    ch = 128
    nch = bpw // ch
    mesh = plsc.VectorSubcoreMesh(core_axis_name="c", subcore_axis_name="s")

    @functools.partial(
        pl.kernel,
        mesh=mesh,
        out_type=jax.ShapeDtypeStruct((nk, d), jnp.float32),
        scratch_types=[
            pltpu.VMEM((ch,), jnp.int32),
            pltpu.VMEM((ch, d), jnp.float32),
            pltpu.SemaphoreType.DMA,
        ],
    )
    def k(table_hbm, idx_hbm, out_hbm, idx_v, rows_v, sem):
        wid = lax.axis_index("s") * ncores + lax.axis_index("c")
        base = wid * bpw

        def body(c, carry):
            off = base + c * ch
            pltpu.sync_copy(idx_hbm.at[pl.ds(off, ch)], idx_v)
            pltpu.async_copy(table_hbm.at[idx_v], rows_v, sem).wait()
            pltpu.sync_copy(rows_v, out_hbm.at[pl.ds(off, ch)])
            return carry

        lax.fori_loop(0, nch, body, 0)

    return k(table, idx)


def _stats_a(g, xqr, xoff, bm=2048):
    nk, gd = g.shape
    grid = nk // bm

    xblk = xoff // 128

    def body(g_ref, xqr_ref, m1_ref, m2_ref):
        @pl.when(pl.program_id(0) == 0)
        def _():
            m1_ref[...] = jnp.zeros_like(m1_ref)
            m2_ref[...] = jnp.zeros_like(m2_ref)

        t = g_ref[:, 0:_K] - xqr_ref[...]
        m1_ref[0:1, :] += jnp.sum(t, axis=0, keepdims=True)
        m2_ref[...] += lax.dot_general(
            t, t, (((0,), (0,)), ((), ())), preferred_element_type=jnp.float32
        )

    return pl.pallas_call(
        body,
        grid=(grid,),
        in_specs=[
            pl.BlockSpec((bm, 128), lambda i: (i, xblk)),
            pl.BlockSpec((bm, _K), lambda i: (i, 0)),
        ],
        out_specs=[
            pl.BlockSpec((8, _K), lambda i: (0, 0)),
            pl.BlockSpec((_K, _K), lambda i: (0, 0)),
        ],
        out_shape=[
            jax.ShapeDtypeStruct((8, _K), jnp.float32),
            jax.ShapeDtypeStruct((_K, _K), jnp.float32),
        ],
        interpret=_INTERP,
    )(g, xqr)


def _pass_b(g, qt, xqr, m1, m2, w1p, pb1, pg1, pbe1, pw2, pb2, cnt, bm=2048):
    nk = g.shape[0]
    co = pw2.shape[1]
    grid = nk // bm
    bq = bm // _K
    inv = 1.0 / cnt

    def body(gkv_ref, gx_ref, qt_ref, xqr_ref, m1_ref, m2_ref, w1_ref, b1_ref,
             g1_ref, be1_ref, w2_ref, b2_ref, w_out, vf_out, sw_ref, sww_ref):
        @pl.when(pl.program_id(0) == 0)
        def _():
            sw_ref[...] = jnp.zeros_like(sw_ref)
            sww_ref[...] = jnp.zeros_like(sww_ref)

        gv = gkv_ref[...]
        kpg = gv[:, :co]
        vpg = gv[:, co:2 * co]
        xg = gx_ref[:, 0:_K]
        w1 = w1_ref[...]
        b1 = b1_ref[...]
        md = m1_ref[0:1, :] * inv
        mu = jnp.dot(md, w1, preferred_element_type=jnp.float32) + b1
        t2 = jnp.dot(m2_ref[...] * inv, w1, preferred_element_type=jnp.float32)
        e2 = jnp.sum(w1 * t2, axis=0, keepdims=True) + b1 * (2.0 * mu - b1)
        var = e2 - mu * mu
        sc = g1_ref[...] / jnp.sqrt(var + _EPS)
        sh = be1_ref[...] - mu * sc

        dd = xg - xqr_ref[...]
        pre = jnp.dot(dd, w1, preferred_element_type=jnp.float32) + b1
        ph = _leaky(pre * sc + sh)
        pe = jnp.dot(ph, w2_ref[...], preferred_element_type=jnp.float32) + b2_ref[...]

        qtb = _rep16(qt_ref[...])
        w = qtb - kpg + pe
        vf = vpg + pe
        w_out[...] = w
        vf_out[...] = vf
        sw_ref[0:1, :] += jnp.sum(w, axis=0, keepdims=True)
        sww_ref[0:1, :] += jnp.sum(w * w, axis=0, keepdims=True)

    return pl.pallas_call(
        body,
        grid=(grid,),
        in_specs=[
            pl.BlockSpec((bm, 2 * co), lambda i: (i, 0)),
            pl.BlockSpec((bm, 128), lambda i: (i, (2 * co) // 128)),
            pl.BlockSpec((bq, co), lambda i: (i, 0)),
            pl.BlockSpec((bm, _K), lambda i: (i, 0)),
            pl.BlockSpec((8, _K), lambda i: (0, 0)),
            pl.BlockSpec((_K, _K), lambda i: (0, 0)),
            pl.BlockSpec((_K, 64), lambda i: (0, 0)),
            pl.BlockSpec((1, 64), lambda i: (0, 0)),
            pl.BlockSpec((1, 64), lambda i: (0, 0)),
            pl.BlockSpec((1, 64), lambda i: (0, 0)),
            pl.BlockSpec((64, co), lambda i: (0, 0)),
            pl.BlockSpec((1, co), lambda i: (0, 0)),
        ],
        out_specs=[
            pl.BlockSpec((bm, co), lambda i: (i, 0)),
            pl.BlockSpec((bm, co), lambda i: (i, 0)),
            pl.BlockSpec((8, co), lambda i: (0, 0)),
            pl.BlockSpec((8, co), lambda i: (0, 0)),
        ],
        out_shape=[
            jax.ShapeDtypeStruct((nk, co), jnp.float32),
            jax.ShapeDtypeStruct((nk, co), jnp.float32),
            jax.ShapeDtypeStruct((8, co), jnp.float32),
            jax.ShapeDtypeStruct((8, co), jnp.float32),
        ],
        interpret=_INTERP,
    )(g, g, qt, xqr, m1, m2, w1p, pb1.reshape(1, -1), pg1.reshape(1, -1),
      pbe1.reshape(1, -1), pw2, pb2.reshape(1, -1))


def _pass_c(w, a1, b1, ww, wb, bm=2048):
    nk, co = w.shape
    grid = nk // bm

    def body(w_ref, a_ref, b_ref, ww_ref, wb_ref, u_out, su_ref, suu_ref):
        @pl.when(pl.program_id(0) == 0)
        def _():
            su_ref[...] = jnp.zeros_like(su_ref)
            suu_ref[...] = jnp.zeros_like(suu_ref)

        h = _leaky(w_ref[...] * a_ref[...] + b_ref[...])
        u = jnp.dot(h, ww_ref[...], preferred_element_type=jnp.float32) + wb_ref[...]
        u_out[...] = u
        su_ref[0:1, :] += jnp.sum(u, axis=0, keepdims=True)
        suu_ref[0:1, :] += jnp.sum(u * u, axis=0, keepdims=True)

    return pl.pallas_call(
        body,
        grid=(grid,),
        in_specs=[
            pl.BlockSpec((bm, co), lambda i: (i, 0)),
            pl.BlockSpec((1, co), lambda i: (0, 0)),
            pl.BlockSpec((1, co), lambda i: (0, 0)),
            pl.BlockSpec((co, co), lambda i: (0, 0)),
            pl.BlockSpec((1, co), lambda i: (0, 0)),
        ],
        out_specs=[
            pl.BlockSpec((bm, co), lambda i: (i, 0)),
            pl.BlockSpec((8, co), lambda i: (0, 0)),
            pl.BlockSpec((8, co), lambda i: (0, 0)),
        ],
        out_shape=[
            jax.ShapeDtypeStruct((nk, co), jnp.float32),
            jax.ShapeDtypeStruct((8, co), jnp.float32),
            jax.ShapeDtypeStruct((8, co), jnp.float32),
        ],
        interpret=_INTERP,
    )(w, a1, b1, ww, wb)


def _pass_d(u, vf, a2, b2, bq=128):
    nk, co = u.shape
    n = nk // _K
    grid = n // bq
    u3 = u.reshape(n, _K, co)
    vf3 = vf.reshape(n, _K, co)

    def body(u_ref, vf_ref, a_ref, b_ref, o_ref):
        a = a_ref[...].reshape(1, 1, co)
        b = b_ref[...].reshape(1, 1, co)
        s = _leaky(u_ref[...] * a + b)
        mx = jnp.max(s, axis=1, keepdims=True)
        e = jnp.exp(s - mx)
        z = jnp.sum(e, axis=1, keepdims=True)
        o_ref[...] = jnp.sum((e / z) * vf_ref[...], axis=1)

    return pl.pallas_call(
        body,
        grid=(grid,),
        in_specs=[
            pl.BlockSpec((bq, _K, co), lambda i: (i, 0, 0)),
            pl.BlockSpec((bq, _K, co), lambda i: (i, 0, 0)),
            pl.BlockSpec((1, co), lambda i: (0, 0)),
            pl.BlockSpec((1, co), lambda i: (0, 0)),
        ],
        out_specs=pl.BlockSpec((bq, co), lambda i: (i, 0)),
        out_shape=jax.ShapeDtypeStruct((n, co), jnp.float32),
        interpret=_INTERP,
    )(u3, vf3, a2, b2)


def kernel(fea_prev, fea_cur, fea_next, xyz_prev, xyz_cur, xyz_next, batch,
           p_w1, p_b1, p_g1, p_be1, p_w2, p_b2,
           q_w, q_b, k_w, k_b, v_w, v_b,
           w_g1, w_be1, w_w, w_wb, w_g2, w_be2):
    n, ci = fea_cur.shape
    co = q_w.shape[1]
    cnt = float(n * _K)

    qt = _mm_bias(fea_cur, q_w, q_b)
    kvw = jnp.concatenate([k_w, v_w], axis=1)
    kvb = jnp.concatenate([k_b, v_b])
    tab_p = _mm_bias(fea_prev, kvw, kvb)
    tab_n = _mm_bias(fea_next, kvw, kvb)

    xq8 = jnp.pad(xyz_cur, ((0, 0), (0, 5)))
    xq16 = jnp.pad(xyz_cur, ((0, 0), (0, _K - 3)))
    xqr = _rep16(xq16)
    w1p = jnp.pad(p_w1, ((0, _K - 3), (0, 0)))

    gs = []
    for xyz_kv, tab in ((xyz_prev, tab_p), (xyz_next, tab_n)):
        xkt = jnp.pad(xyz_kv, ((0, 0), (0, 5))).T
        idx = _knn(xq8, xkt)
        x128 = jnp.pad(xyz_kv, ((0, 0), (0, 125)))
        table = jnp.concatenate([tab, x128], axis=1)
        gs.append(_sc_gather(table, idx.reshape(-1)))

    outs = []
    for g in gs:
        m1, m2 = _stats_a(g, xqr, 2 * co)
        w, vf, sw, sww = _pass_b(g, qt, xqr, m1, m2, w1p,
                                 p_b1, p_g1, p_be1, p_w2, p_b2, cnt)
        swt = jnp.sum(sw, axis=0, keepdims=True)
        swwt = jnp.sum(sww, axis=0, keepdims=True)
        mean_w = swt / cnt
        var_w = swwt / cnt - mean_w * mean_w
        a1 = w_g1.reshape(1, -1) / jnp.sqrt(var_w + _EPS)
        b1 = w_be1.reshape(1, -1) - mean_w * a1
        u, su, suu = _pass_c(w, a1, b1, w_w, w_wb.reshape(1, -1))
        sut = jnp.sum(su, axis=0, keepdims=True)
        suut = jnp.sum(suu, axis=0, keepdims=True)
        mean_u = sut / cnt
        var_u = suut / cnt - mean_u * mean_u
        a2 = w_g2.reshape(1, -1) / jnp.sqrt(var_u + _EPS)
        b2 = w_be2.reshape(1, -1) - mean_u * a2
        outs.append(_pass_d(u, vf, a2, b2))
    return (outs[0], outs[1])


# speedup vs baseline: 4.9107x; 1.0003x over previous
import functools

import jax
import jax.numpy as jnp
from jax import lax
from jax.experimental import pallas as pl
from jax.experimental.pallas import tpu as pltpu
from jax.experimental.pallas import tpu_sc as plsc

_K = 16
_EPS = 1e-5
_SLOPE = 0.01


def _leaky(x):
    return jnp.where(x >= 0, x, _SLOPE * x)


def _rep16(x):
    m, c = x.shape
    return jnp.broadcast_to(x[:, None, :], (m, _K, c)).reshape(m * _K, c)


def _mm_bias(x, w, b, bm=1024):
    m, kd = x.shape
    nd = w.shape[1]

    def body(x_ref, w_ref, b_ref, o_ref):
        o_ref[...] = (
            jnp.dot(x_ref[...], w_ref[...], preferred_element_type=jnp.float32)
            + b_ref[...]
        )

    return pl.pallas_call(
        body,
        grid=(m // bm,),
        in_specs=[
            pl.BlockSpec((bm, kd), lambda i: (i, 0)),
            pl.BlockSpec((kd, nd), lambda i: (0, 0)),
            pl.BlockSpec((1, nd), lambda i: (0, 0)),
        ],
        out_specs=pl.BlockSpec((bm, nd), lambda i: (i, 0)),
        out_shape=jax.ShapeDtypeStruct((m, nd), jnp.float32),
    )(x, w, b.reshape(1, -1))


def _knn(yq8, xkt, tq=256):
    nq = yq8.shape[0]
    nk = xkt.shape[1]

    def body(yq_ref, xk_ref, o_ref, dscr):
        yq = yq_ref[...]
        xk = xk_ref[...]
        ny = jnp.sum(yq * yq, axis=1, keepdims=True)
        nx = jnp.sum(xk * xk, axis=0, keepdims=True)
        d = ny + nx - 2.0 * jnp.dot(yq, xk, preferred_element_type=jnp.float32)
        dscr[...] = d
        li = lax.broadcasted_iota(jnp.int32, (tq, _K), 1)
        big = jnp.int32(0x7FFFFFFF)

        def it(j, acc):
            dv = dscr[...]
            col = lax.broadcasted_iota(jnp.int32, dv.shape, 1)
            mn = jnp.min(dv, axis=1)
            cand = jnp.where(dv == mn[:, None], col, big)
            c = jnp.min(cand, axis=1)
            dscr[...] = jnp.where(col == c[:, None], jnp.float32(jnp.inf), dv)
            return jnp.where(li == j, c[:, None], acc)

        o_ref[...] = lax.fori_loop(0, _K, it, jnp.zeros((tq, _K), jnp.int32))

    return pl.pallas_call(
        body,
        grid=(nq // tq,),
        in_specs=[
            pl.BlockSpec((tq, 8), lambda i: (i, 0)),
            pl.BlockSpec((8, nk), lambda i: (0, 0)),
        ],
        out_specs=pl.BlockSpec((tq, _K), lambda i: (i, 0)),
        out_shape=jax.ShapeDtypeStruct((nq, _K), jnp.int32),
        scratch_shapes=[pltpu.VMEM((tq, nk), jnp.float32)],
    )(yq8, xkt)


def _sc_gather(table, idx):
    nk = idx.shape[0]
    d = table.shape[1]
    ncores, nsub = 2, 16
    nw = ncores * nsub
    bpw = nk // nw
    ch = 128
    nch = bpw // ch
    mesh = plsc.VectorSubcoreMesh(core_axis_name="c", subcore_axis_name="s")

    @functools.partial(
        pl.kernel,
        mesh=mesh,
        out_type=jax.ShapeDtypeStruct((nk, d), jnp.float32),
        scratch_types=[
            pltpu.VMEM((ch,), jnp.int32),
            pltpu.VMEM((ch, d), jnp.float32),
            pltpu.SemaphoreType.DMA,
        ],
    )
    def k(table_hbm, idx_hbm, out_hbm, idx_v, rows_v, sem):
        wid = lax.axis_index("s") * ncores + lax.axis_index("c")
        base = wid * bpw

        def body(c, carry):
            off = base + c * ch
            pltpu.sync_copy(idx_hbm.at[pl.ds(off, ch)], idx_v)
            pltpu.async_copy(table_hbm.at[idx_v], rows_v, sem).wait()
            pltpu.sync_copy(rows_v, out_hbm.at[pl.ds(off, ch)])
            return carry

        lax.fori_loop(0, nch, body, 0)

    return k(table, idx)


def _stats_a(g, xqr, xoff, bm=2048):
    nk, gd = g.shape
    grid = nk // bm

    xblk = xoff // 128

    def body(g_ref, xqr_ref, m1_ref, m2_ref):
        @pl.when(pl.program_id(0) == 0)
        def _():
            m1_ref[...] = jnp.zeros_like(m1_ref)
            m2_ref[...] = jnp.zeros_like(m2_ref)

        t = g_ref[:, 0:_K] - xqr_ref[...]
        m1_ref[0:1, :] += jnp.sum(t, axis=0, keepdims=True)
        m2_ref[...] += lax.dot_general(
            t, t, (((0,), (0,)), ((), ())), preferred_element_type=jnp.float32
        )

    return pl.pallas_call(
        body,
        grid=(grid,),
        in_specs=[
            pl.BlockSpec((bm, 128), lambda i: (i, xblk)),
            pl.BlockSpec((bm, _K), lambda i: (i, 0)),
        ],
        out_specs=[
            pl.BlockSpec((8, _K), lambda i: (0, 0)),
            pl.BlockSpec((_K, _K), lambda i: (0, 0)),
        ],
        out_shape=[
            jax.ShapeDtypeStruct((8, _K), jnp.float32),
            jax.ShapeDtypeStruct((_K, _K), jnp.float32),
        ],
    )(g, xqr)


def _pass_b(g, qt, xqr, m1, m2, w1p, pb1, pg1, pbe1, pw2, pb2, cnt, bm=2048):
    nk = g.shape[0]
    co = pw2.shape[1]
    grid = nk // bm
    bq = bm // _K
    inv = 1.0 / cnt

    def body(gkv_ref, gx_ref, qt_ref, xqr_ref, m1_ref, m2_ref, w1_ref, b1_ref,
             g1_ref, be1_ref, w2_ref, b2_ref, w_out, vf_out, sw_ref, sww_ref):
        @pl.when(pl.program_id(0) == 0)
        def _():
            sw_ref[...] = jnp.zeros_like(sw_ref)
            sww_ref[...] = jnp.zeros_like(sww_ref)

        gv = gkv_ref[...]
        kpg = gv[:, :co]
        vpg = gv[:, co:2 * co]
        xg = gx_ref[:, 0:_K]
        w1 = w1_ref[...]
        b1 = b1_ref[...]
        md = m1_ref[0:1, :] * inv
        mu = jnp.dot(md, w1, preferred_element_type=jnp.float32) + b1
        t2 = jnp.dot(m2_ref[...] * inv, w1, preferred_element_type=jnp.float32)
        e2 = jnp.sum(w1 * t2, axis=0, keepdims=True) + b1 * (2.0 * mu - b1)
        var = e2 - mu * mu
        sc = g1_ref[...] / jnp.sqrt(var + _EPS)
        sh = be1_ref[...] - mu * sc

        dd = xg - xqr_ref[...]
        pre = jnp.dot(dd, w1, preferred_element_type=jnp.float32) + b1
        ph = _leaky(pre * sc + sh)
        pe = jnp.dot(ph, w2_ref[...], preferred_element_type=jnp.float32) + b2_ref[...]

        qtb = _rep16(qt_ref[...])
        w = qtb - kpg + pe
        vf = vpg + pe
        w_out[...] = w
        vf_out[...] = vf
        sw_ref[0:1, :] += jnp.sum(w, axis=0, keepdims=True)
        sww_ref[0:1, :] += jnp.sum(w * w, axis=0, keepdims=True)

    return pl.pallas_call(
        body,
        grid=(grid,),
        in_specs=[
            pl.BlockSpec((bm, 2 * co), lambda i: (i, 0)),
            pl.BlockSpec((bm, 128), lambda i: (i, (2 * co) // 128)),
            pl.BlockSpec((bq, co), lambda i: (i, 0)),
            pl.BlockSpec((bm, _K), lambda i: (i, 0)),
            pl.BlockSpec((8, _K), lambda i: (0, 0)),
            pl.BlockSpec((_K, _K), lambda i: (0, 0)),
            pl.BlockSpec((_K, 64), lambda i: (0, 0)),
            pl.BlockSpec((1, 64), lambda i: (0, 0)),
            pl.BlockSpec((1, 64), lambda i: (0, 0)),
            pl.BlockSpec((1, 64), lambda i: (0, 0)),
            pl.BlockSpec((64, co), lambda i: (0, 0)),
            pl.BlockSpec((1, co), lambda i: (0, 0)),
        ],
        out_specs=[
            pl.BlockSpec((bm, co), lambda i: (i, 0)),
            pl.BlockSpec((bm, co), lambda i: (i, 0)),
            pl.BlockSpec((8, co), lambda i: (0, 0)),
            pl.BlockSpec((8, co), lambda i: (0, 0)),
        ],
        out_shape=[
            jax.ShapeDtypeStruct((nk, co), jnp.float32),
            jax.ShapeDtypeStruct((nk, co), jnp.float32),
            jax.ShapeDtypeStruct((8, co), jnp.float32),
            jax.ShapeDtypeStruct((8, co), jnp.float32),
        ],
    )(g, g, qt, xqr, m1, m2, w1p, pb1.reshape(1, -1), pg1.reshape(1, -1),
      pbe1.reshape(1, -1), pw2, pb2.reshape(1, -1))


def _pass_c(w, a1, b1, ww, wb, bm=2048):
    nk, co = w.shape
    grid = nk // bm

    def body(w_ref, a_ref, b_ref, ww_ref, wb_ref, u_out, su_ref, suu_ref):
        @pl.when(pl.program_id(0) == 0)
        def _():
            su_ref[...] = jnp.zeros_like(su_ref)
            suu_ref[...] = jnp.zeros_like(suu_ref)

        h = _leaky(w_ref[...] * a_ref[...] + b_ref[...])
        u = jnp.dot(h, ww_ref[...], preferred_element_type=jnp.float32) + wb_ref[...]
        u_out[...] = u
        su_ref[0:1, :] += jnp.sum(u, axis=0, keepdims=True)
        suu_ref[0:1, :] += jnp.sum(u * u, axis=0, keepdims=True)

    return pl.pallas_call(
        body,
        grid=(grid,),
        in_specs=[
            pl.BlockSpec((bm, co), lambda i: (i, 0)),
            pl.BlockSpec((1, co), lambda i: (0, 0)),
            pl.BlockSpec((1, co), lambda i: (0, 0)),
            pl.BlockSpec((co, co), lambda i: (0, 0)),
            pl.BlockSpec((1, co), lambda i: (0, 0)),
        ],
        out_specs=[
            pl.BlockSpec((bm, co), lambda i: (i, 0)),
            pl.BlockSpec((8, co), lambda i: (0, 0)),
            pl.BlockSpec((8, co), lambda i: (0, 0)),
        ],
        out_shape=[
            jax.ShapeDtypeStruct((nk, co), jnp.float32),
            jax.ShapeDtypeStruct((8, co), jnp.float32),
            jax.ShapeDtypeStruct((8, co), jnp.float32),
        ],
    )(w, a1, b1, ww, wb)


def _pass_d(u, vf, a2, b2, bq=128):
    nk, co = u.shape
    n = nk // _K
    grid = n // bq
    u3 = u.reshape(n, _K, co)
    vf3 = vf.reshape(n, _K, co)

    def body(u_ref, vf_ref, a_ref, b_ref, o_ref):
        a = a_ref[...].reshape(1, 1, co)
        b = b_ref[...].reshape(1, 1, co)
        s = _leaky(u_ref[...] * a + b)
        mx = jnp.max(s, axis=1, keepdims=True)
        e = jnp.exp(s - mx)
        z = jnp.sum(e, axis=1, keepdims=True)
        o_ref[...] = jnp.sum((e / z) * vf_ref[...], axis=1)

    return pl.pallas_call(
        body,
        grid=(grid,),
        in_specs=[
            pl.BlockSpec((bq, _K, co), lambda i: (i, 0, 0)),
            pl.BlockSpec((bq, _K, co), lambda i: (i, 0, 0)),
            pl.BlockSpec((1, co), lambda i: (0, 0)),
            pl.BlockSpec((1, co), lambda i: (0, 0)),
        ],
        out_specs=pl.BlockSpec((bq, co), lambda i: (i, 0)),
        out_shape=jax.ShapeDtypeStruct((n, co), jnp.float32),
    )(u3, vf3, a2, b2)


def kernel(fea_prev, fea_cur, fea_next, xyz_prev, xyz_cur, xyz_next, batch,
           p_w1, p_b1, p_g1, p_be1, p_w2, p_b2,
           q_w, q_b, k_w, k_b, v_w, v_b,
           w_g1, w_be1, w_w, w_wb, w_g2, w_be2):
    n, ci = fea_cur.shape
    co = q_w.shape[1]
    cnt = float(n * _K)

    qt = _mm_bias(fea_cur, q_w, q_b)
    kvw = jnp.concatenate([k_w, v_w], axis=1)
    kvb = jnp.concatenate([k_b, v_b])
    tab_p = _mm_bias(fea_prev, kvw, kvb)
    tab_n = _mm_bias(fea_next, kvw, kvb)

    xq8 = jnp.pad(xyz_cur, ((0, 0), (0, 5)))
    xq16 = jnp.pad(xyz_cur, ((0, 0), (0, _K - 3)))
    xqr = _rep16(xq16)
    w1p = jnp.pad(p_w1, ((0, _K - 3), (0, 0)))

    gs = []
    for xyz_kv, tab in ((xyz_prev, tab_p), (xyz_next, tab_n)):
        xkt = jnp.pad(xyz_kv, ((0, 0), (0, 5))).T
        idx = _knn(xq8, xkt)
        x128 = jnp.pad(xyz_kv, ((0, 0), (0, 125)))
        table = jnp.concatenate([tab, x128], axis=1)
        gs.append(_sc_gather(table, idx.reshape(-1)))

    outs = []
    for g in gs:
        m1, m2 = _stats_a(g, xqr, 2 * co)
        w, vf, sw, sww = _pass_b(g, qt, xqr, m1, m2, w1p,
                                 p_b1, p_g1, p_be1, p_w2, p_b2, cnt)
        swt = jnp.sum(sw, axis=0, keepdims=True)
        swwt = jnp.sum(sww, axis=0, keepdims=True)
        mean_w = swt / cnt
        var_w = swwt / cnt - mean_w * mean_w
        a1 = w_g1.reshape(1, -1) / jnp.sqrt(var_w + _EPS)
        b1 = w_be1.reshape(1, -1) - mean_w * a1
        u, su, suu = _pass_c(w, a1, b1, w_w, w_wb.reshape(1, -1))
        sut = jnp.sum(su, axis=0, keepdims=True)
        suut = jnp.sum(suu, axis=0, keepdims=True)
        mean_u = sut / cnt
        var_u = suut / cnt - mean_u * mean_u
        a2 = w_g2.reshape(1, -1) / jnp.sqrt(var_u + _EPS)
        b2 = w_be2.reshape(1, -1) - mean_u * a2
        outs.append(_pass_d(u, vf, a2, b2))
    return (outs[0], outs[1])
